# Initial kernel scaffold; baseline (speedup 1.0000x reference)
import jax
import jax.numpy as jnp
from jax.experimental import pallas as pl


def kernel(noise, edge_attr, edge_index, fc1_w, fc1_b, Wq, bq, Wk, bk, Wv, bv, We, be, Wskip, bskip, atom_w, atom_b, other_w, other_b, edge_w, edge_b):
    raise NotImplementedError("write your pallas kernel here")


# speedup vs baseline: 97.2580x; 97.2580x over previous
import numpy as np
import jax
import jax.numpy as jnp
from jax.experimental import pallas as pl

B = 64
NN = 32
NOISE = 128
HID = 128
HEADS = 4
NEF = 4
NLAYERS = 4
NATOM = 9
NNF = 16
N = B * NN
E = B * NN * (NN - 1)
G = 4
NG = B // G
ROWS = G * NN
SCALE = 1.0 / np.sqrt(HID)


def _fused_kernel(noise_ref, a_ref, fc1_w_ref, fc1_b_ref,
                  wq_ref, bq_ref, wk_ref, bk_ref, wv_ref, bv_ref,
                  we_ref, be_ref, wskip_ref, bskip_ref,
                  atom_w_ref, atom_b_ref, other_w_ref, other_b_ref,
                  edge_w_ref, edge_b_ref,
                  node_out_ref, edge_out_ref):
    f32 = jnp.float32

    nz = noise_ref[0]
    h = jnp.maximum(nz @ fc1_w_ref[...] + fc1_b_ref[...], 0.0)
    r4 = jax.lax.broadcasted_iota(jnp.int32, (ROWS, G), 0)
    c4 = jax.lax.broadcasted_iota(jnp.int32, (ROWS, G), 1)
    sel = (r4 // NN == c4).astype(f32)
    x = sel @ h

    ri = jax.lax.broadcasted_iota(jnp.int32, (ROWS, ROWS), 0)
    ci = jax.lax.broadcasted_iota(jnp.int32, (ROWS, ROWS), 1)
    mask = (ri // NN == ci // NN) & (ri != ci)
    neg = f32(-1e30)

    for l in range(NLAYERS):
        q = x @ wq_ref[l] + bq_ref[l:l + 1, :]
        k = x @ wk_ref[l] + bk_ref[l:l + 1, :]
        v = x @ wv_ref[l] + bv_ref[l:l + 1, :]
        we_l = we_ref[l]
        be_l = be_ref[l]

        hacc = jnp.zeros((ROWS, HID), f32)
        for hd in range(HEADS):
            sl = slice(hd * HID, (hd + 1) * HID)
            qh = q[:, sl]
            kh = k[:, sl]
            vh = v[:, sl]
            we_h = we_l[:, sl]
            be_h = be_l[sl]

            s = jax.lax.dot_general(qh, kh, (((1,), (1,)), ((), ())))
            qe = jax.lax.dot_general(qh, we_h, (((1,), (1,)), ((), ())))
            for f in range(NEF):
                s = s + qe[:, f:f + 1] * a_ref[0, f]
            qbe = jnp.sum(qh * be_h[None, :], axis=1, keepdims=True)
            s = (s + qbe) * f32(SCALE)
            s = jnp.where(mask, s, neg)

            smax = jnp.max(s, axis=1, keepdims=True)
            ex = jnp.exp(s - smax)
            attn = ex / jnp.sum(ex, axis=1, keepdims=True)

            m = jax.lax.dot_general(attn, vh, (((1,), (0,)), ((), ())))
            for f in range(NEF):
                waf = jnp.sum(attn * a_ref[0, f], axis=1, keepdims=True)
                m = m + waf * we_h[f:f + 1, :]
            m = m + be_h[None, :]
            hacc = hacc + m

        out = hacc * f32(1.0 / HEADS) + x @ wskip_ref[l] + bskip_ref[l:l + 1, :]
        x = jnp.maximum(out, 0.0)

    al = x @ atom_w_ref[...] + atom_b_ref[...]
    amax = jnp.max(al, axis=1, keepdims=True)
    aex = jnp.exp(al - amax)
    ap = aex / jnp.sum(aex, axis=1, keepdims=True)
    ot = jax.nn.sigmoid(x @ other_w_ref[...] + other_b_ref[...])
    node_out_ref[...] = jax.nn.sigmoid(jnp.concatenate([ap, ot], axis=1))

    ew = edge_w_ref[...]
    ef1 = x @ ew[:HID] + edge_b_ref[...]
    ef2t = jax.lax.dot_general(ew[HID:], x, (((0,), (1,)), ((), ())))
    for g in range(G):
        gs = slice(g * NN, (g + 1) * NN)
        for f in range(NEF):
            col = ef1[gs, f:f + 1]
            row = ef2t[f:f + 1, gs]
            edge_out_ref[g, f] = jax.nn.sigmoid(col + row)


def kernel(noise, edge_attr, edge_index, fc1_w, fc1_b, Wq, bq, Wk, bk, Wv, bv,
           We, be, Wskip, bskip, atom_w, atom_b, other_w, other_b,
           edge_w, edge_b):
    del edge_index

    er = edge_attr.reshape(B, NN, NN - 1, NEF).transpose(0, 3, 1, 2)
    er = er.reshape(B, NEF, NN - 1, NN)
    er = jnp.pad(er, ((0, 0), (0, 0), (0, 0), (0, 1)))
    er = er.reshape(B, NEF, NN * NN - 1)
    er = jnp.pad(er, ((0, 0), (0, 0), (1, 0)))
    dense = er.reshape(B, NEF, NN, NN)
    dense_r = dense.transpose(0, 1, 3, 2).reshape(NG, G, NEF, NN, NN)
    a_big = jnp.zeros((NG, NEF, ROWS, ROWS), jnp.float32)
    for g in range(G):
        gs = slice(g * NN, (g + 1) * NN)
        a_big = a_big.at[:, :, gs, gs].set(dense_r[:, g])

    noise_r = noise.reshape(NG, G, NOISE)
    fc1_b2 = fc1_b.reshape(1, HID)
    atom_b2 = atom_b.reshape(1, NATOM)
    other_b2 = other_b.reshape(1, NNF - 1)
    edge_b2 = edge_b.reshape(1, NEF)

    def c2(i): return (0, 0)
    def c3(i): return (0, 0, 0)

    in_specs = [
        pl.BlockSpec((1, G, NOISE), lambda i: (i, 0, 0)),
        pl.BlockSpec((1, NEF, ROWS, ROWS), lambda i: (i, 0, 0, 0)),
        pl.BlockSpec((NOISE, HID), c2),
        pl.BlockSpec((1, HID), c2),
        pl.BlockSpec((NLAYERS, HID, HEADS * HID), c3),
        pl.BlockSpec((NLAYERS, HEADS * HID), c2),
        pl.BlockSpec((NLAYERS, HID, HEADS * HID), c3),
        pl.BlockSpec((NLAYERS, HEADS * HID), c2),
        pl.BlockSpec((NLAYERS, HID, HEADS * HID), c3),
        pl.BlockSpec((NLAYERS, HEADS * HID), c2),
        pl.BlockSpec((NLAYERS, NEF, HEADS * HID), c3),
        pl.BlockSpec((NLAYERS, HEADS * HID), c2),
        pl.BlockSpec((NLAYERS, HID, HID), c3),
        pl.BlockSpec((NLAYERS, HID), c2),
        pl.BlockSpec((HID, NATOM), c2),
        pl.BlockSpec((1, NATOM), c2),
        pl.BlockSpec((HID, NNF - 1), c2),
        pl.BlockSpec((1, NNF - 1), c2),
        pl.BlockSpec((2 * HID, NEF), c2),
        pl.BlockSpec((1, NEF), c2),
    ]
    out_specs = [
        pl.BlockSpec((ROWS, NATOM + NNF - 1), lambda i: (i, 0)),
        pl.BlockSpec((G, NEF, NN, NN), lambda i: (i, 0, 0, 0)),
    ]
    out_shape = [
        jax.ShapeDtypeStruct((N, NATOM + NNF - 1), jnp.float32),
        jax.ShapeDtypeStruct((B, NEF, NN, NN), jnp.float32),
    ]

    node_features, edge_dense = pl.pallas_call(
        _fused_kernel,
        grid=(NG,),
        in_specs=in_specs,
        out_specs=out_specs,
        out_shape=out_shape,
    )(noise_r, a_big, fc1_w, fc1_b2, Wq, bq, Wk, bk, Wv, bv, We, be,
      Wskip, bskip, atom_w, atom_b2, other_w, other_b2, edge_w, edge_b2)

    flat = edge_dense.reshape(B, NEF, NN * NN)[:, :, 1:]
    flat = flat.reshape(B, NEF, NN - 1, NN + 1)[:, :, :, :-1]
    off = flat.reshape(B, NEF, NN, NN - 1)
    edge_features = off.transpose(0, 2, 3, 1).reshape(E, NEF)
    return node_features, edge_features


# speedup vs baseline: 137.8669x; 1.4175x over previous
import numpy as np
import jax
import jax.numpy as jnp
from jax.experimental import pallas as pl
from jax.experimental.pallas import tpu as pltpu

B = 64
NN = 32
NOISE = 128
HID = 128
HEADS = 4
NEF = 4
NLAYERS = 4
NATOM = 9
NNF = 16
N = B * NN
E = B * NN * (NN - 1)
G = 4
NG = B // G
ROWS = G * NN
SCALE = 1.0 / np.sqrt(HID)


def _fused_kernel(noise_ref, a_ref, fc1_w_ref, fc1_b_ref,
                  wq_ref, bq_ref, wk_ref, bk_ref, wv_ref, bv_ref,
                  we_ref, be_ref, wskip_ref, bskip_ref,
                  atom_w_ref, atom_b_ref, other_w_ref, other_b_ref,
                  edge_w_ref, edge_b_ref,
                  node_out_ref, edge_out_ref):
    f32 = jnp.float32

    nz = noise_ref[0]
    h = jnp.maximum(nz @ fc1_w_ref[...] + fc1_b_ref[...], 0.0)
    r4 = jax.lax.broadcasted_iota(jnp.int32, (ROWS, G), 0)
    c4 = jax.lax.broadcasted_iota(jnp.int32, (ROWS, G), 1)
    sel = (r4 // NN == c4).astype(f32)
    x = sel @ h

    ri = jax.lax.broadcasted_iota(jnp.int32, (ROWS, ROWS), 0)
    ci = jax.lax.broadcasted_iota(jnp.int32, (ROWS, ROWS), 1)
    mask = (ri // NN == ci // NN) & (ri != ci)
    neg = f32(-1e30)

    a_planes = []
    for f in range(NEF):
        rows = []
        for g in range(G):
            pieces = []
            if g > 0:
                pieces.append(jnp.zeros((NN, NN * g), f32))
            pieces.append(a_ref[0, g, f])
            if g < G - 1:
                pieces.append(jnp.zeros((NN, NN * (G - 1 - g)), f32))
            rows.append(pieces[0] if len(pieces) == 1
                        else jnp.concatenate(pieces, axis=1))
        a_planes.append(jnp.concatenate(rows, axis=0))

    for l in range(NLAYERS):
        q = x @ wq_ref[l] + bq_ref[l:l + 1, :]
        k = x @ wk_ref[l] + bk_ref[l:l + 1, :]
        v = x @ wv_ref[l] + bv_ref[l:l + 1, :]
        we_l = we_ref[l]
        be_l = be_ref[l:l + 1, :]

        hacc = jnp.zeros((ROWS, HID), f32)
        for hd in range(HEADS):
            sl = slice(hd * HID, (hd + 1) * HID)
            qh = q[:, sl]
            kh = k[:, sl]
            vh = v[:, sl]
            we_h = we_l[:, sl]
            be_h = be_l[:, sl]

            s = jax.lax.dot_general(kh, qh, (((1,), (1,)), ((), ())))
            qet = jax.lax.dot_general(we_h, qh, (((1,), (1,)), ((), ())))
            for f in range(NEF):
                s = s + a_planes[f] * qet[f:f + 1, :]
            qbe = jax.lax.dot_general(be_h, qh, (((1,), (1,)), ((), ())))
            s = (s + qbe) * f32(SCALE)
            s = jnp.where(mask, s, neg)

            smax = jnp.max(s, axis=0, keepdims=True)
            ex = jnp.exp(s - smax)
            attn = ex / jnp.sum(ex, axis=0, keepdims=True)

            m = jax.lax.dot_general(attn, vh, (((0,), (0,)), ((), ())))
            wa = jnp.concatenate(
                [jnp.sum(attn * a_planes[f], axis=0, keepdims=True)
                 for f in range(NEF)], axis=0)
            m = m + jax.lax.dot_general(wa, we_h, (((0,), (0,)), ((), ())))
            m = m + be_h
            hacc = hacc + m

        out = hacc * f32(1.0 / HEADS) + x @ wskip_ref[l] + bskip_ref[l:l + 1, :]
        x = jnp.maximum(out, 0.0)

    al = x @ atom_w_ref[...] + atom_b_ref[...]
    amax = jnp.max(al, axis=1, keepdims=True)
    aex = jnp.exp(al - amax)
    ap = aex / jnp.sum(aex, axis=1, keepdims=True)
    ot = jax.nn.sigmoid(x @ other_w_ref[...] + other_b_ref[...])
    node_out_ref[...] = jax.nn.sigmoid(jnp.concatenate([ap, ot], axis=1))

    ew = edge_w_ref[...]
    ef1 = x @ ew[:HID] + edge_b_ref[...]
    ef2 = x @ ew[HID:]
    for g in range(G):
        gs = slice(g * NN, (g + 1) * NN)
        src = ef1[gs].reshape(NN, 1, NEF)
        dst = ef2[gs].reshape(1, NN, NEF)
        edge_out_ref[g] = jax.nn.sigmoid(src + dst)


def kernel(noise, edge_attr, edge_index, fc1_w, fc1_b, Wq, bq, Wk, bk, Wv, bv,
           We, be, Wskip, bskip, atom_w, atom_b, other_w, other_b,
           edge_w, edge_b):
    del edge_index

    er = edge_attr.reshape(B, NN, NN - 1, NEF).transpose(0, 3, 1, 2)
    er = er.reshape(B, NEF, NN - 1, NN)
    er = jnp.pad(er, ((0, 0), (0, 0), (0, 0), (0, 1)))
    er = er.reshape(B, NEF, NN * NN - 1)
    er = jnp.pad(er, ((0, 0), (0, 0), (1, 0)))
    a_dense = er.reshape(NG, G, NEF, NN, NN)

    noise_r = noise.reshape(NG, G, NOISE)
    fc1_b2 = fc1_b.reshape(1, HID)
    atom_b2 = atom_b.reshape(1, NATOM)
    other_b2 = other_b.reshape(1, NNF - 1)
    edge_b2 = edge_b.reshape(1, NEF)

    def c2(i): return (0, 0)
    def c3(i): return (0, 0, 0)

    in_specs = [
        pl.BlockSpec((1, G, NOISE), lambda i: (i, 0, 0)),
        pl.BlockSpec((1, G, NEF, NN, NN), lambda i: (i, 0, 0, 0, 0)),
        pl.BlockSpec((NOISE, HID), c2),
        pl.BlockSpec((1, HID), c2),
        pl.BlockSpec((NLAYERS, HID, HEADS * HID), c3),
        pl.BlockSpec((NLAYERS, HEADS * HID), c2),
        pl.BlockSpec((NLAYERS, HID, HEADS * HID), c3),
        pl.BlockSpec((NLAYERS, HEADS * HID), c2),
        pl.BlockSpec((NLAYERS, HID, HEADS * HID), c3),
        pl.BlockSpec((NLAYERS, HEADS * HID), c2),
        pl.BlockSpec((NLAYERS, NEF, HEADS * HID), c3),
        pl.BlockSpec((NLAYERS, HEADS * HID), c2),
        pl.BlockSpec((NLAYERS, HID, HID), c3),
        pl.BlockSpec((NLAYERS, HID), c2),
        pl.BlockSpec((HID, NATOM), c2),
        pl.BlockSpec((1, NATOM), c2),
        pl.BlockSpec((HID, NNF - 1), c2),
        pl.BlockSpec((1, NNF - 1), c2),
        pl.BlockSpec((2 * HID, NEF), c2),
        pl.BlockSpec((1, NEF), c2),
    ]
    out_specs = [
        pl.BlockSpec((ROWS, NATOM + NNF - 1), lambda i: (i, 0)),
        pl.BlockSpec((G, NN, NN, NEF), lambda i: (i, 0, 0, 0)),
    ]
    out_shape = [
        jax.ShapeDtypeStruct((N, NATOM + NNF - 1), jnp.float32),
        jax.ShapeDtypeStruct((B, NN, NN, NEF), jnp.float32),
    ]

    node_features, edge_dense = pl.pallas_call(
        _fused_kernel,
        grid=(NG,),
        in_specs=in_specs,
        out_specs=out_specs,
        out_shape=out_shape,
        compiler_params=pltpu.CompilerParams(
            dimension_semantics=("parallel",)),
    )(noise_r, a_dense, fc1_w, fc1_b2, Wq, bq, Wk, bk, Wv, bv, We, be,
      Wskip, bskip, atom_w, atom_b2, other_w, other_b2, edge_w, edge_b2)

    flat = edge_dense.reshape(B, NN * NN, NEF)[:, 1:, :]
    flat = flat.reshape(B, NN - 1, NN + 1, NEF)[:, :, :NN, :]
    edge_features = flat.reshape(E, NEF)
    return node_features, edge_features
